# Initial kernel scaffold; baseline (speedup 1.0000x reference)
import jax
import jax.numpy as jnp
from jax.experimental import pallas as pl


def kernel(tri_idx, barycentric, vertex_color, tri_buf):
    raise NotImplementedError("write your pallas kernel here")


# speedup vs baseline: 24.9692x; 24.9692x over previous
import jax
import jax.numpy as jnp
from jax import lax
from jax.experimental import pallas as pl
from jax.experimental.pallas import tpu as pltpu
from jax.experimental.pallas import tpu_sc as plsc
import functools

N_PIX = 262144
N_TRI = 3968
N_VTX = 1986
TEX_CH = 3

NC = 2
NS = 16
LANES = 16
NW = NC * NS
PIX_PER_W = N_PIX // NW
CHUNK = 2048
N_SUB = PIX_PER_W // CHUNK
GROUPS = CHUNK // LANES


def _sc_body(tidx_hbm, bary_hbm, vc_hbm, tri_hbm, out_hbm,
             vc_v, tri_v, tidx_v, bary_v, out_v):
    wid = lax.axis_index("s") * NC + lax.axis_index("c")
    pltpu.sync_copy(vc_hbm, vc_v)
    pltpu.sync_copy(tri_hbm, tri_v)
    lane = lax.iota(jnp.int32, 16)
    lane9 = lane * 9

    def sub_body(s, carry):
        base = wid * PIX_PER_W + s * CHUNK
        for k in range(3):
            pltpu.sync_copy(tidx_hbm.at[pl.ds(k * N_PIX + base, CHUNK)],
                            tidx_v.at[pl.ds(k * CHUNK, CHUNK)])
            pltpu.sync_copy(bary_hbm.at[pl.ds(k * N_PIX + base, CHUNK)],
                            bary_v.at[pl.ds(k * CHUNK, CHUNK)])

        def grp_body(g, gcarry):
            offs = g * LANES
            acc = [None] * 9
            for k in range(3):
                t = tidx_v[pl.ds(k * CHUNK + offs, LANES)]
                w = bary_v[pl.ds(k * CHUNK + offs, LANES)]
                t3 = t * 3
                for j in range(3):
                    vtx = plsc.load_gather(tri_v, [t3 + j])
                    v3 = vtx * 3
                    for c in range(3):
                        val = plsc.load_gather(vc_v, [v3 + c])
                        o = 3 * j + c
                        term = val * w
                        acc[o] = term if k == 0 else acc[o] + term
            rowbase = offs * 9
            for o in range(9):
                plsc.store_scatter(out_v, [rowbase + lane9 + o], acc[o])
            return gcarry

        lax.fori_loop(0, GROUPS, grp_body, 0, unroll=False)
        pltpu.sync_copy(out_v, out_hbm.at[pl.ds(base * 9, CHUNK * 9)])
        return carry

    lax.fori_loop(0, N_SUB, sub_body, 0, unroll=False)


@jax.jit
def _tri_mesh_sc(tidx, bary, vc, tri):
    mesh = plsc.VectorSubcoreMesh(
        core_axis_name="c", subcore_axis_name="s",
        num_cores=NC, num_subcores=NS)
    out_flat = pl.kernel(
        _sc_body,
        out_type=jax.ShapeDtypeStruct((N_PIX * 9,), jnp.float32),
        mesh=mesh,
        compiler_params=pltpu.CompilerParams(needs_layout_passes=False),
        scratch_types=[
            pltpu.VMEM((N_VTX * TEX_CH,), jnp.float32),
            pltpu.VMEM((N_TRI * 3,), jnp.int32),
            pltpu.VMEM((3 * CHUNK,), jnp.int32),
            pltpu.VMEM((3 * CHUNK,), jnp.float32),
            pltpu.VMEM((CHUNK * 9,), jnp.float32),
        ],
    )(tidx, bary, vc, tri)
    return out_flat.reshape(N_PIX, 3, TEX_CH)


def kernel(tri_idx, barycentric, vertex_color, tri_buf):
    bary = barycentric.reshape(3 * N_PIX)
    return _tri_mesh_sc(tri_idx.reshape(3 * N_PIX), bary,
                        vertex_color.reshape(N_VTX * TEX_CH),
                        tri_buf.reshape(N_TRI * 3))


# speedup vs baseline: 140.2981x; 5.6188x over previous
import jax
import jax.numpy as jnp
from jax import lax
from jax.experimental import pallas as pl
from jax.experimental.pallas import tpu as pltpu
from jax.experimental.pallas import tpu_sc as plsc
import functools

N_PIX = 262144
N_TRI = 3968
N_VTX = 1986
TEX_CH = 3

NC = 2
NS = 16
LANES = 16
NW = NC * NS
PIX_PER_W = N_PIX // NW
CHUNK = 2048
N_SUB = PIX_PER_W // CHUNK
GROUPS = CHUNK // LANES


def _sc_body(tidx_hbm, bary_hbm, vc_hbm, tri_hbm, out_hbm,
             vc_v, tri_v, tidx_v, bary_v, out_v):
    wid = lax.axis_index("s") * NC + lax.axis_index("c")
    pltpu.sync_copy(vc_hbm, vc_v)
    pltpu.sync_copy(tri_hbm, tri_v)

    def sub_body(s, carry):
        base = wid * PIX_PER_W + s * CHUNK
        for k in range(3):
            pltpu.sync_copy(tidx_hbm.at[pl.ds(k * N_PIX + base, CHUNK)],
                            tidx_v.at[pl.ds(k * CHUNK, CHUNK)])
            pltpu.sync_copy(bary_hbm.at[pl.ds(k * N_PIX + base, CHUNK)],
                            bary_v.at[pl.ds(k * CHUNK, CHUNK)])

        def grp_body(g, gcarry):
            offs = g * LANES
            acc = [None] * 9
            for k in range(3):
                t = tidx_v[pl.ds(k * CHUNK + offs, LANES)]
                w = bary_v[pl.ds(k * CHUNK + offs, LANES)]
                t3 = t * 3
                for j in range(3):
                    vtx = plsc.load_gather(tri_v, [t3 + j])
                    v3 = vtx * 3
                    for c in range(3):
                        val = plsc.load_gather(vc_v, [v3 + c])
                        o = 3 * j + c
                        term = val * w
                        acc[o] = term if k == 0 else acc[o] + term
            for o in range(9):
                out_v[pl.ds(o * CHUNK + offs, LANES)] = acc[o]
            return gcarry

        lax.fori_loop(0, GROUPS, grp_body, 0, unroll=False)
        for o in range(9):
            pltpu.sync_copy(out_v.at[pl.ds(o * CHUNK, CHUNK)],
                            out_hbm.at[pl.ds(o * N_PIX + base, CHUNK)])
        return carry

    lax.fori_loop(0, N_SUB, sub_body, 0, unroll=False)


@jax.jit
def _tri_mesh_sc(tidx, bary, vc, tri):
    mesh = plsc.VectorSubcoreMesh(
        core_axis_name="c", subcore_axis_name="s",
        num_cores=NC, num_subcores=NS)
    out_flat = pl.kernel(
        _sc_body,
        out_type=jax.ShapeDtypeStruct((9 * N_PIX,), jnp.float32),
        mesh=mesh,
        compiler_params=pltpu.CompilerParams(needs_layout_passes=False),
        scratch_types=[
            pltpu.VMEM((N_VTX * TEX_CH,), jnp.float32),
            pltpu.VMEM((N_TRI * 3,), jnp.int32),
            pltpu.VMEM((3 * CHUNK,), jnp.int32),
            pltpu.VMEM((3 * CHUNK,), jnp.float32),
            pltpu.VMEM((9 * CHUNK,), jnp.float32),
        ],
    )(tidx, bary, vc, tri)
    return out_flat.reshape(3, TEX_CH, N_PIX).transpose(2, 0, 1)


def kernel(tri_idx, barycentric, vertex_color, tri_buf):
    bary = barycentric.reshape(3 * N_PIX)
    return _tri_mesh_sc(tri_idx.reshape(3 * N_PIX), bary,
                        vertex_color.reshape(N_VTX * TEX_CH),
                        tri_buf.reshape(N_TRI * 3))
